# Initial kernel scaffold; baseline (speedup 1.0000x reference)
import jax
import jax.numpy as jnp
from jax.experimental import pallas as pl


def kernel(x, edge_index, Wl0, Wr0, b0, Wl1, Wr1, b1, Wl2, Wr2, b2):
    raise NotImplementedError("write your pallas kernel here")


# speedup vs baseline: 3.2545x; 3.2545x over previous
import jax
import jax.numpy as jnp
from jax import lax
from jax.experimental import pallas as pl
from jax.experimental.pallas import tpu as pltpu
from jax.experimental.pallas import tpu_sc as plsc

_DEBUG_LEVEL = 6

N = 10000
E = 320000
D = 128

NC = 2
NS = 16
NW = NC * NS

CH = 128
CPT = 79
EPT = CPT * CH
EPAD = NW * EPT

NP = 10112
RPT = NP // NS
_RCHUNKS = ((0, 128), (128, 128), (256, 128), (384, 128), (512, 120))

RB = 1264


def _sc_agg_body(h, srcp, dstp, zrows, out_acc, sidx, didx, rows, acc_sh, sem):
    cid = lax.axis_index("c")
    sid = lax.axis_index("s")
    wid = cid * NS + sid

    pltpu.sync_copy(zrows.at[pl.ds(0, CH)], rows)
    for o_, s_ in _RCHUNKS:
        r0 = sid * RPT + o_
        pltpu.sync_copy(rows.at[pl.ds(0, s_)], acc_sh.at[pl.ds(r0, s_)])
    if _DEBUG_LEVEL >= 4:
        plsc.subcore_barrier()

    base = wid * EPT

    def chunk(c, carry):
        off = pl.multiple_of(base + c * CH, CH)
        pltpu.sync_copy(srcp.at[pl.ds(off, CH)], sidx)
        pltpu.sync_copy(dstp.at[pl.ds(off, CH)], didx)
        pltpu.async_copy(h.at[sidx], rows, sem).wait()
        pltpu.sync_copy(rows, acc_sh.at[didx], add=True)
        return carry

    if _DEBUG_LEVEL >= 6:
        lax.fori_loop(0, CPT, chunk, 0)
    if _DEBUG_LEVEL >= 4:
        plsc.subcore_barrier()

    for o_, s_ in _RCHUNKS:
        r0 = sid * RPT + o_
        if _DEBUG_LEVEL >= 5:
            pltpu.sync_copy(acc_sh.at[pl.ds(r0, s_)], rows.at[pl.ds(0, s_)])
        pltpu.sync_copy(rows.at[pl.ds(0, s_)],
                        out_acc.at[pl.ds(cid * NP + r0, s_)])


_sc_agg = pl.kernel(
    _sc_agg_body,
    out_type=[jax.ShapeDtypeStruct((NC * NP, D), jnp.float32)],
    mesh=plsc.VectorSubcoreMesh(core_axis_name="c", subcore_axis_name="s"),
    scratch_types=[
        pltpu.VMEM((CH,), jnp.int32),
        pltpu.VMEM((CH,), jnp.int32),
        pltpu.VMEM((CH, D), jnp.float32),
        pltpu.VMEM_SHARED((NP, D), jnp.float32),
        pltpu.SemaphoreType.DMA,
    ],
)


def _combine(acc, degp, h, Wl, Wr, b, relu):
    def body(p0r, p1r, d0r, d1r, hr, wlr, wrr, br, o):
        deg = d0r[:, :1] + d1r[:, :1]
        inv = 1.0 / jnp.maximum(deg, 1.0)
        mean = (p0r[...] + p1r[...]) * inv
        out = jnp.dot(mean, wlr[...], preferred_element_type=jnp.float32)
        out = out + jnp.dot(hr[...], wrr[...], preferred_element_type=jnp.float32)
        out = out + br[...]
        if relu:
            out = jnp.maximum(out, 0.0)
        o[...] = out

    p0, p1 = acc[:NP], acc[NP:]
    d0, d1 = degp[:NP], degp[NP:]
    return pl.pallas_call(
        body,
        grid=(NP // RB,),
        in_specs=[
            pl.BlockSpec((RB, D), lambda i: (i, 0)),
            pl.BlockSpec((RB, D), lambda i: (i, 0)),
            pl.BlockSpec((RB, D), lambda i: (i, 0)),
            pl.BlockSpec((RB, D), lambda i: (i, 0)),
            pl.BlockSpec((RB, D), lambda i: (i, 0)),
            pl.BlockSpec((D, D), lambda i: (0, 0)),
            pl.BlockSpec((D, D), lambda i: (0, 0)),
            pl.BlockSpec((1, D), lambda i: (0, 0)),
        ],
        out_specs=pl.BlockSpec((RB, D), lambda i: (i, 0)),
        out_shape=jax.ShapeDtypeStruct((NP, D), jnp.float32),
    )(p0, p1, d0, d1, h, Wl, Wr, b.reshape(1, D))


def kernel(x, edge_index, Wl0, Wr0, b0, Wl1, Wr1, b1, Wl2, Wr2, b2):
    src = edge_index[0].astype(jnp.int32)
    dst = edge_index[1].astype(jnp.int32)
    pad = EPAD - E
    srcp = jnp.concatenate([src, jnp.zeros((pad,), jnp.int32)])
    dstp = jnp.concatenate([dst, jnp.full((pad,), N, jnp.int32)])
    hp = jnp.concatenate([x, jnp.zeros((NP - N, D), jnp.float32)], axis=0)

    zrows = jnp.zeros((CH, D), jnp.float32)
    ones = jnp.ones((NP, D), jnp.float32)

    (degp,) = _sc_agg(ones, srcp, dstp, zrows)

    (acc0,) = _sc_agg(hp, srcp, dstp, zrows)
    h1 = _combine(acc0, degp, hp, Wl0, Wr0, b0, relu=True)
    (acc1,) = _sc_agg(h1, srcp, dstp, zrows)
    h2 = _combine(acc1, degp, h1, Wl1, Wr1, b1, relu=True)
    (acc2,) = _sc_agg(h2, srcp, dstp, zrows)
    h3 = _combine(acc2, degp, h2, Wl2, Wr2, b2, relu=False)
    return h3[:N]


# speedup vs baseline: 3.4210x; 1.0511x over previous
import jax
import jax.numpy as jnp
from jax import lax
from jax.experimental import pallas as pl
from jax.experimental.pallas import tpu as pltpu
from jax.experimental.pallas import tpu_sc as plsc

N = 10000
E = 320000
D = 128

NC = 2
NS = 16
NW = NC * NS

CH = 128
CPT = 80
EPT = CPT * CH
EPAD = NW * EPT

NP = 10112
RPT = NP // NS
_RCHUNKS = ((0, 128), (128, 128), (256, 128), (384, 128), (512, 120))

RB = 1264


def _zero_acc(zrows, rows, acc_sh, sid):
    pltpu.sync_copy(zrows, rows)
    for o_, s_ in _RCHUNKS:
        r0 = sid * RPT + o_
        pltpu.sync_copy(rows.at[pl.ds(0, s_)], acc_sh.at[pl.ds(r0, s_)])


def _copy_out(out_acc, rows, acc_sh, cid, sid):
    for o_, s_ in _RCHUNKS:
        r0 = sid * RPT + o_
        pltpu.sync_copy(acc_sh.at[pl.ds(r0, s_)], rows.at[pl.ds(0, s_)])
        pltpu.sync_copy(rows.at[pl.ds(0, s_)],
                        out_acc.at[pl.ds(cid * NP + r0, s_)])


def _sc_agg_body(h, edges, zrows, out_acc,
                 eidx0, eidx1, rows0, rows1, acc_sh, sem0, sem1):
    cid = lax.axis_index("c")
    sid = lax.axis_index("s")
    wid = cid * NS + sid

    _zero_acc(zrows, rows0, acc_sh, sid)
    plsc.subcore_barrier()

    base = wid * CPT

    def load(k, eidx):
        pltpu.sync_copy(edges.at[pl.ds(pl.multiple_of(2 * k, 2), 2)], eidx)

    load(base, eidx0)
    g0 = pltpu.async_copy(h.at[eidx0.at[0]], rows0, sem0)

    def body(i, carry):
        k1 = base + 2 * i + 1
        load(k1, eidx1)
        g1 = pltpu.async_copy(h.at[eidx1.at[0]], rows1, sem1)
        pltpu.make_async_copy(h.at[eidx0.at[0]], rows0, sem0).wait()
        pltpu.sync_copy(rows0, acc_sh.at[eidx0.at[1]], add=True)
        load(k1 + 1, eidx0)
        pltpu.async_copy(h.at[eidx0.at[0]], rows0, sem0)
        g1.wait()
        pltpu.sync_copy(rows1, acc_sh.at[eidx1.at[1]], add=True)
        return carry

    lax.fori_loop(0, CPT // 2 - 1, body, 0)
    pltpu.make_async_copy(h.at[eidx0.at[0]], rows0, sem0).wait()
    pltpu.sync_copy(rows0, acc_sh.at[eidx0.at[1]], add=True)
    load(base + CPT - 1, eidx1)
    pltpu.async_copy(h.at[eidx1.at[0]], rows1, sem1).wait()
    pltpu.sync_copy(rows1, acc_sh.at[eidx1.at[1]], add=True)

    plsc.subcore_barrier()
    _copy_out(out_acc, rows0, acc_sh, cid, sid)


_sc_agg = pl.kernel(
    _sc_agg_body,
    out_type=[jax.ShapeDtypeStruct((NC * NP, D), jnp.float32)],
    mesh=plsc.VectorSubcoreMesh(core_axis_name="c", subcore_axis_name="s"),
    scratch_types=[
        pltpu.VMEM((2, CH), jnp.int32),
        pltpu.VMEM((2, CH), jnp.int32),
        pltpu.VMEM((CH, D), jnp.float32),
        pltpu.VMEM((CH, D), jnp.float32),
        pltpu.VMEM_SHARED((NP, D), jnp.float32),
        pltpu.SemaphoreType.DMA,
        pltpu.SemaphoreType.DMA,
    ],
)


def _sc_deg_body(edges, zrows, ones_hbm, out_acc,
                 didx0, didx1, onesv, rows, acc_sh):
    cid = lax.axis_index("c")
    sid = lax.axis_index("s")
    wid = cid * NS + sid

    _zero_acc(zrows, rows, acc_sh, sid)
    pltpu.sync_copy(ones_hbm, onesv)
    plsc.subcore_barrier()

    base = wid * CPT

    def load_dst(k, didx):
        pltpu.sync_copy(
            edges.at[pl.ds(pl.multiple_of(2 * k, 2) + 1, 1)], didx)

    def body(i, carry):
        load_dst(base + 2 * i, didx0)
        pltpu.sync_copy(onesv, acc_sh.at[didx0.at[0]], add=True)
        load_dst(base + 2 * i + 1, didx1)
        pltpu.sync_copy(onesv, acc_sh.at[didx1.at[0]], add=True)
        return carry

    lax.fori_loop(0, CPT // 2, body, 0)

    plsc.subcore_barrier()
    _copy_out(out_acc, rows, acc_sh, cid, sid)


_sc_deg = pl.kernel(
    _sc_deg_body,
    out_type=[jax.ShapeDtypeStruct((NC * NP, D), jnp.float32)],
    mesh=plsc.VectorSubcoreMesh(core_axis_name="c", subcore_axis_name="s"),
    scratch_types=[
        pltpu.VMEM((1, CH), jnp.int32),
        pltpu.VMEM((1, CH), jnp.int32),
        pltpu.VMEM((CH, D), jnp.float32),
        pltpu.VMEM((CH, D), jnp.float32),
        pltpu.VMEM_SHARED((NP, D), jnp.float32),
    ],
)


def _combine(acc, degp, h, Wl, Wr, b, relu):
    def body(p0r, p1r, d0r, d1r, hr, wlr, wrr, br, o):
        deg = d0r[:, :1] + d1r[:, :1]
        inv = 1.0 / jnp.maximum(deg, 1.0)
        mean = (p0r[...] + p1r[...]) * inv
        out = jnp.dot(mean, wlr[...], preferred_element_type=jnp.float32)
        out = out + jnp.dot(hr[...], wrr[...], preferred_element_type=jnp.float32)
        out = out + br[...]
        if relu:
            out = jnp.maximum(out, 0.0)
        o[...] = out

    p0, p1 = acc[:NP], acc[NP:]
    d0, d1 = degp[:NP], degp[NP:]
    return pl.pallas_call(
        body,
        grid=(NP // RB,),
        in_specs=[
            pl.BlockSpec((RB, D), lambda i: (i, 0)),
            pl.BlockSpec((RB, D), lambda i: (i, 0)),
            pl.BlockSpec((RB, D), lambda i: (i, 0)),
            pl.BlockSpec((RB, D), lambda i: (i, 0)),
            pl.BlockSpec((RB, D), lambda i: (i, 0)),
            pl.BlockSpec((D, D), lambda i: (0, 0)),
            pl.BlockSpec((D, D), lambda i: (0, 0)),
            pl.BlockSpec((1, D), lambda i: (0, 0)),
        ],
        out_specs=pl.BlockSpec((RB, D), lambda i: (i, 0)),
        out_shape=jax.ShapeDtypeStruct((NP, D), jnp.float32),
    )(p0, p1, d0, d1, h, Wl, Wr, b.reshape(1, D))


def kernel(x, edge_index, Wl0, Wr0, b0, Wl1, Wr1, b1, Wl2, Wr2, b2):
    src = edge_index[0].astype(jnp.int32)
    dst = edge_index[1].astype(jnp.int32)
    pad = EPAD - E
    srcp = jnp.concatenate([src, jnp.zeros((pad,), jnp.int32)])
    dstp = jnp.concatenate([dst, jnp.full((pad,), N, jnp.int32)])
    edges = jnp.stack(
        [srcp.reshape(-1, CH), dstp.reshape(-1, CH)], axis=1
    ).reshape(-1, CH)
    hp = jnp.concatenate([x, jnp.zeros((NP - N, D), jnp.float32)], axis=0)

    zrows = jnp.zeros((CH, D), jnp.float32)
    ones = jnp.ones((CH, D), jnp.float32)

    (degp,) = _sc_deg(edges, zrows, ones)
    (acc0,) = _sc_agg(hp, edges, zrows)
    h1 = _combine(acc0, degp, hp, Wl0, Wr0, b0, relu=True)
    (acc1,) = _sc_agg(h1, edges, zrows)
    h2 = _combine(acc1, degp, h1, Wl1, Wr1, b1, relu=True)
    (acc2,) = _sc_agg(h2, edges, zrows)
    h3 = _combine(acc2, degp, h2, Wl2, Wr2, b2, relu=False)
    return h3[:N]
